# Initial kernel scaffold; baseline (speedup 1.0000x reference)
import jax
import jax.numpy as jnp
from jax import lax
from jax.experimental import pallas as pl
from jax.experimental.pallas import tpu as pltpu
from jax.experimental.pallas import tpu_sc as plsc

N_MODEL_ROWS = 2_000_000
N_POINTS = 500_000
NC = 2
NS = 16
LANES = 16

N_PAD = 500_224
PER_TILE = N_PAD // NS
ROWS_PER_TILE = N_MODEL_ROWS // NS


def _sc_body(ids_hbm, x_hbm, y_hbm, m_hbm, acc_in_hbm, den_in_hbm,
             acc_out_hbm, den_out_hbm, idx_v, x_v, y_v, m_v, shared):
    c = lax.axis_index("c")
    s = lax.axis_index("s")
    base = s * PER_TILE
    mbase = s * ROWS_PER_TILE

    @pl.when(c == 0)
    def _():
        pltpu.sync_copy(acc_in_hbm.at[pl.ds(mbase, ROWS_PER_TILE)],
                        shared.at[pl.ds(mbase, ROWS_PER_TILE)])

    @pl.when(c == 1)
    def _():
        pltpu.sync_copy(den_in_hbm.at[pl.ds(mbase, ROWS_PER_TILE)],
                        shared.at[pl.ds(mbase, ROWS_PER_TILE)])

    pltpu.sync_copy(ids_hbm.at[pl.ds(base, PER_TILE)], idx_v)
    pltpu.sync_copy(m_hbm.at[pl.ds(base, PER_TILE)], m_v)

    @pl.when(c == 0)
    def _():
        pltpu.sync_copy(x_hbm.at[pl.ds(base, PER_TILE)], x_v)
        pltpu.sync_copy(y_hbm.at[pl.ds(base, PER_TILE)], y_v)

        def step(i, carry):
            sl = pl.ds(i * LANES, LANES)
            xx = x_v[sl]
            yy = y_v[sl]
            ss = xx * xx + yy * yy
            x_v[sl] = jnp.sqrt(ss) * m_v[sl]
            return carry

        lax.fori_loop(0, PER_TILE // LANES, step, 0)

    plsc.subcore_barrier()

    @pl.when(c == 0)
    def _():
        pltpu.sync_copy(x_v, shared.at[idx_v], add=True)

    @pl.when(c == 1)
    def _():
        pltpu.sync_copy(m_v, shared.at[idx_v], add=True)

    plsc.subcore_barrier()

    @pl.when(c == 0)
    def _():
        pltpu.sync_copy(shared.at[pl.ds(mbase, ROWS_PER_TILE)],
                        acc_out_hbm.at[pl.ds(mbase, ROWS_PER_TILE)])

    @pl.when(c == 1)
    def _():
        pltpu.sync_copy(shared.at[pl.ds(mbase, ROWS_PER_TILE)],
                        den_out_hbm.at[pl.ds(mbase, ROWS_PER_TILE)])


_sc_call = pl.kernel(
    _sc_body,
    out_type=(
        jax.ShapeDtypeStruct((N_MODEL_ROWS,), jnp.float32),
        jax.ShapeDtypeStruct((N_MODEL_ROWS,), jnp.float32),
    ),
    mesh=plsc.VectorSubcoreMesh(
        core_axis_name="c", subcore_axis_name="s",
        num_cores=NC, num_subcores=NS),
    scratch_types=[
        pltpu.VMEM((PER_TILE,), jnp.int32),
        pltpu.VMEM((PER_TILE,), jnp.float32),
        pltpu.VMEM((PER_TILE,), jnp.float32),
        pltpu.VMEM((PER_TILE,), jnp.float32),
        pltpu.VMEM_SHARED((N_MODEL_ROWS,), jnp.float32),
    ],
)


@jax.jit
def kernel(viewspace_grad, update_filter, gaussian_ids, xyz_gradient_accum,
           denom):
    pad = N_PAD - N_POINTS
    x = jnp.pad(viewspace_grad[:, 0], (0, pad))
    y = jnp.pad(viewspace_grad[:, 1], (0, pad))
    m = jnp.pad(update_filter.astype(jnp.float32), (0, pad))
    ids = jnp.pad(gaussian_ids.astype(jnp.int32), (0, pad))
    acc, den = _sc_call(ids, x, y, m,
                        xyz_gradient_accum[:, 0], denom[:, 0])
    return jnp.stack([acc, den], axis=-1)


# speedup vs baseline: 4.0757x; 4.0757x over previous
import jax
import jax.numpy as jnp
from jax import lax
from jax.experimental import pallas as pl
from jax.experimental.pallas import tpu as pltpu
from jax.experimental.pallas import tpu_sc as plsc

N_MODEL_ROWS = 2_000_000
N_POINTS = 500_000
NC = 2
NS = 16
LANES = 16

N_PAD = 524_288
TC_ROWS = N_PAD // 128
TC_BLOCK = 1024
PER_TILE = N_PAD // NS
ROWS_PER_CORE = N_MODEL_ROWS // NC
DRAIN_TILES = 8
ROWS_PER_DRAIN = ROWS_PER_CORE // DRAIN_TILES
CHUNK = 25_000
N_CHUNKS = ROWS_PER_DRAIN // CHUNK
DUMMY_ROWS = 4096


def _norm_body(x_ref, y_ref, m_ref, o_ref):
    xx = x_ref[...]
    yy = y_ref[...]
    o_ref[...] = jnp.sqrt(xx * xx + yy * yy) * m_ref[...]


_norm_call = pl.pallas_call(
    _norm_body,
    out_shape=jax.ShapeDtypeStruct((TC_ROWS, 128), jnp.float32),
    grid=(TC_ROWS // TC_BLOCK,),
    in_specs=[pl.BlockSpec((TC_BLOCK, 128), lambda i: (i, 0))] * 3,
    out_specs=pl.BlockSpec((TC_BLOCK, 128), lambda i: (i, 0)),
)


def _sc_body(ids_hbm, contrib_hbm, m_hbm, acc_in_hbm, den_in_hbm,
             acc_out_hbm, den_out_hbm, idx_v, val_v, shared):
    c = lax.axis_index("c")
    s = lax.axis_index("s")
    base = s * PER_TILE
    lo = c * ROWS_PER_CORE

    pltpu.sync_copy(ids_hbm.at[pl.ds(base, PER_TILE)], idx_v)

    def idx_step(i, carry):
        sl = pl.ds(i * LANES, LANES)
        li = idx_v[sl] - lo
        neg1 = -(li >> 31)
        high1 = -((ROWS_PER_CORE - 1 - li) >> 31)
        f = neg1 + high1
        dummy = ROWS_PER_CORE + (li & (DUMMY_ROWS - 1))
        idx_v[sl] = li + f * (dummy - li)
        return carry

    lax.fori_loop(0, PER_TILE // LANES, idx_step, 0)

    def _init_from(src_hbm):
        @pl.when(s < DRAIN_TILES)
        def _():
            for j in range(N_CHUNKS):
                o = s * ROWS_PER_DRAIN + j * CHUNK
                pltpu.sync_copy(src_hbm.at[pl.ds(lo + o, CHUNK)],
                                val_v.at[pl.ds(0, CHUNK)])
                pltpu.sync_copy(val_v.at[pl.ds(0, CHUNK)],
                                shared.at[pl.ds(o, CHUNK)])

    def _drain_to(dst_hbm):
        @pl.when(s < DRAIN_TILES)
        def _():
            for j in range(N_CHUNKS):
                o = s * ROWS_PER_DRAIN + j * CHUNK
                pltpu.sync_copy(shared.at[pl.ds(o, CHUNK)],
                                val_v.at[pl.ds(0, CHUNK)])
                pltpu.sync_copy(val_v.at[pl.ds(0, CHUNK)],
                                dst_hbm.at[pl.ds(lo + o, CHUNK)])

    def _scatter_serialized():
        for r in range(NS):
            @pl.when(s == r)
            def _():
                pltpu.sync_copy(val_v, shared.at[idx_v], add=True)
            plsc.subcore_barrier()

    _init_from(acc_in_hbm)
    pltpu.sync_copy(contrib_hbm.at[pl.ds(base, PER_TILE)], val_v)
    plsc.subcore_barrier()
    _scatter_serialized()
    _drain_to(acc_out_hbm)
    plsc.subcore_barrier()

    _init_from(den_in_hbm)
    pltpu.sync_copy(m_hbm.at[pl.ds(base, PER_TILE)], val_v)
    plsc.subcore_barrier()
    _scatter_serialized()
    _drain_to(den_out_hbm)


_sc_call = pl.kernel(
    _sc_body,
    out_type=(
        jax.ShapeDtypeStruct((N_MODEL_ROWS,), jnp.float32),
        jax.ShapeDtypeStruct((N_MODEL_ROWS,), jnp.float32),
    ),
    mesh=plsc.VectorSubcoreMesh(
        core_axis_name="c", subcore_axis_name="s",
        num_cores=NC, num_subcores=NS),
    scratch_types=[
        pltpu.VMEM((PER_TILE,), jnp.int32),
        pltpu.VMEM((PER_TILE,), jnp.float32),
        pltpu.VMEM_SHARED((ROWS_PER_CORE + DUMMY_ROWS,), jnp.float32),
    ],
)


@jax.jit
def kernel(viewspace_grad, update_filter, gaussian_ids, xyz_gradient_accum,
           denom):
    pad = N_PAD - N_POINTS
    x = jnp.pad(viewspace_grad[:, 0], (0, pad)).reshape(TC_ROWS, 128)
    y = jnp.pad(viewspace_grad[:, 1], (0, pad)).reshape(TC_ROWS, 128)
    m = jnp.pad(update_filter.astype(jnp.float32), (0, pad))
    ids = jnp.pad(gaussian_ids.astype(jnp.int32), (0, pad))
    contrib = _norm_call(x, y, m.reshape(TC_ROWS, 128)).reshape(N_PAD)
    acc, den = _sc_call(ids, contrib, m,
                        xyz_gradient_accum[:, 0], denom[:, 0])
    return jnp.stack([acc, den], axis=-1)


# speedup vs baseline: 5.9708x; 1.4650x over previous
import jax
import jax.numpy as jnp
from jax import lax
from jax.experimental import pallas as pl
from jax.experimental.pallas import tpu as pltpu
from jax.experimental.pallas import tpu_sc as plsc

N_MODEL_ROWS = 2_000_000
N_POINTS = 500_000
NC = 2
NS = 16
LANES = 16
SENTINEL = 2 * N_MODEL_ROWS

N_PAD = 524_288
TC_ROWS = N_PAD // 128
TC_BLOCK = 1024
PER_TILE = N_PAD // NS
ROWS_PER_CORE = N_MODEL_ROWS // NC
DRAIN_TILES = 8
ROWS_PER_DRAIN = ROWS_PER_CORE // DRAIN_TILES
CHUNK = 25_000
N_CHUNKS = ROWS_PER_DRAIN // CHUNK
DUMMY_ROWS = 64
WIN = 2048


def _norm_body(x_ref, y_ref, m_ref, ids_ref, o_ref, ids2_ref):
    xx = x_ref[...]
    yy = y_ref[...]
    m = m_ref[...]
    o_ref[...] = jnp.sqrt(xx * xx + yy * yy) * m.astype(jnp.float32)
    ids2_ref[...] = jnp.where(m, ids_ref[...], SENTINEL)


_norm_call = pl.pallas_call(
    _norm_body,
    out_shape=(
        jax.ShapeDtypeStruct((TC_ROWS, 128), jnp.float32),
        jax.ShapeDtypeStruct((TC_ROWS, 128), jnp.int32),
    ),
    grid=(TC_ROWS // TC_BLOCK,),
    in_specs=[pl.BlockSpec((TC_BLOCK, 128), lambda i: (i, 0))] * 4,
    out_specs=[pl.BlockSpec((TC_BLOCK, 128), lambda i: (i, 0))] * 2,
)


def _sc_body(ids_hbm, contrib_hbm, acc_in_hbm, den_in_hbm,
             acc_out_hbm, den_out_hbm, idx_v, val_v, shared):
    c = lax.axis_index("c")
    s = lax.axis_index("s")
    base = s * PER_TILE
    lo = c * ROWS_PER_CORE

    def _init_from(src_hbm):
        @pl.when(s < DRAIN_TILES)
        def _():
            for j in range(N_CHUNKS):
                o = s * ROWS_PER_DRAIN + j * CHUNK
                pltpu.sync_copy(src_hbm.at[pl.ds(lo + o, CHUNK)],
                                val_v.at[pl.ds(0, CHUNK)])
                pltpu.sync_copy(val_v.at[pl.ds(0, CHUNK)],
                                shared.at[pl.ds(o, CHUNK)])

    def _drain_to(dst_hbm):
        @pl.when(s < DRAIN_TILES)
        def _():
            for j in range(N_CHUNKS):
                o = s * ROWS_PER_DRAIN + j * CHUNK
                pltpu.sync_copy(shared.at[pl.ds(o, CHUNK)],
                                val_v.at[pl.ds(0, CHUNK)])
                pltpu.sync_copy(val_v.at[pl.ds(0, CHUNK)],
                                dst_hbm.at[pl.ds(lo + o, CHUNK)])

    pltpu.sync_copy(ids_hbm.at[pl.ds(base, PER_TILE)],
                    idx_v.at[pl.ds(0, PER_TILE)])
    _init_from(acc_in_hbm)

    lane = lax.iota(jnp.int32, LANES)
    dummy_vec = ROWS_PER_CORE + lane

    def rstep(i, carry):
        sl = pl.ds(i * LANES, LANES)
        li = idx_v[sl] - lo
        inr = (li >= 0) & (li < ROWS_PER_CORE)
        idx_v[sl] = jnp.where(inr, li, dummy_vec)
        return carry

    lax.fori_loop(0, PER_TILE // LANES, rstep, 0)

    pltpu.sync_copy(contrib_hbm.at[pl.ds(base, PER_TILE)],
                    val_v.at[pl.ds(0, PER_TILE)])
    plsc.subcore_barrier()

    def _scatter():
        def wstep(w, carry):
            sl = pl.ds(w * WIN, WIN)
            pltpu.sync_copy(val_v.at[sl], shared.at[idx_v.at[sl]], add=True)
            return carry
        lax.fori_loop(0, PER_TILE // WIN, wstep, 0)

    _scatter()
    plsc.subcore_barrier()
    _drain_to(acc_out_hbm)
    plsc.subcore_barrier()

    _init_from(den_in_hbm)

    def ones_step(i, carry):
        val_v[pl.ds(i * LANES, LANES)] = jnp.full((LANES,), 1.0, jnp.float32)
        return carry

    lax.fori_loop(0, PER_TILE // LANES, ones_step, 0)
    plsc.subcore_barrier()
    _scatter()
    plsc.subcore_barrier()
    _drain_to(den_out_hbm)


_sc_call = pl.kernel(
    _sc_body,
    out_type=(
        jax.ShapeDtypeStruct((N_MODEL_ROWS,), jnp.float32),
        jax.ShapeDtypeStruct((N_MODEL_ROWS,), jnp.float32),
    ),
    mesh=plsc.VectorSubcoreMesh(
        core_axis_name="c", subcore_axis_name="s",
        num_cores=NC, num_subcores=NS),
    scratch_types=[
        pltpu.VMEM((PER_TILE,), jnp.int32),
        pltpu.VMEM((PER_TILE,), jnp.float32),
        pltpu.VMEM_SHARED((ROWS_PER_CORE + DUMMY_ROWS,), jnp.float32),
    ],
)


@jax.jit
def kernel(viewspace_grad, update_filter, gaussian_ids, xyz_gradient_accum,
           denom):
    pad = N_PAD - N_POINTS
    x = jnp.pad(viewspace_grad[:, 0], (0, pad)).reshape(TC_ROWS, 128)
    y = jnp.pad(viewspace_grad[:, 1], (0, pad)).reshape(TC_ROWS, 128)
    m = jnp.pad(update_filter, (0, pad)).reshape(TC_ROWS, 128)
    ids = jnp.pad(gaussian_ids.astype(jnp.int32), (0, pad))
    contrib, ids2 = _norm_call(x, y, m, ids.reshape(TC_ROWS, 128))
    acc, den = _sc_call(ids2.reshape(N_PAD), contrib.reshape(N_PAD),
                        xyz_gradient_accum[:, 0], denom[:, 0])
    return jnp.stack([acc, den], axis=-1)


# speedup vs baseline: 7.8259x; 1.3107x over previous
import jax
import jax.numpy as jnp
from jax import lax
from jax.experimental import pallas as pl
from jax.experimental.pallas import tpu as pltpu
from jax.experimental.pallas import tpu_sc as plsc

N_MODEL_ROWS = 2_000_000
N_POINTS = 500_000
NC = 2
NS = 16
LANES = 16

N_PAD = 524_288
TC_ROWS = N_PAD // 128
TC_BLOCK = 1024
PER_TILE = N_PAD // NS
ROWS_PER_CORE = N_MODEL_ROWS // NC
BCH = 25_000
N_BCH = ROWS_PER_CORE // BCH
DUMMY_ROWS = 64
WIN = 2048


def _norm_body(x_ref, y_ref, m_ref, ids_ref, o_ref, ida_ref, idb_ref):
    xx = x_ref[...]
    yy = y_ref[...]
    m = m_ref[...]
    ids = ids_ref[...]
    o_ref[...] = jnp.sqrt(xx * xx + yy * yy) * m.astype(jnp.float32)
    sink = ROWS_PER_CORE + (
        lax.broadcasted_iota(jnp.int32, (TC_BLOCK, 128), 1) % DUMMY_ROWS)
    ida_ref[...] = jnp.where(m & (ids < ROWS_PER_CORE), ids, sink)
    idb_ref[...] = jnp.where(m & (ids >= ROWS_PER_CORE),
                             ids - ROWS_PER_CORE, sink)


_norm_call = pl.pallas_call(
    _norm_body,
    out_shape=(
        jax.ShapeDtypeStruct((TC_ROWS, 128), jnp.float32),
        jax.ShapeDtypeStruct((TC_ROWS, 128), jnp.int32),
        jax.ShapeDtypeStruct((TC_ROWS, 128), jnp.int32),
    ),
    grid=(TC_ROWS // TC_BLOCK,),
    in_specs=[pl.BlockSpec((TC_BLOCK, 128), lambda i: (i, 0))] * 4,
    out_specs=[pl.BlockSpec((TC_BLOCK, 128), lambda i: (i, 0))] * 3,
)


def _sc_body(ida_hbm, idb_hbm, contrib_hbm, acc_in_hbm, den_in_hbm,
             acc_out_hbm, den_out_hbm, idx_v, val_v, shared):
    c = lax.axis_index("c")
    s = lax.axis_index("s")
    base = s * PER_TILE
    lo = c * ROWS_PER_CORE

    def _init_from(src_hbm):
        for j in range((N_BCH + NS - 1) // NS):
            k = j * NS + s
            @pl.when(k < N_BCH)
            def _():
                o = k * BCH
                pltpu.sync_copy(src_hbm.at[pl.ds(lo + o, BCH)],
                                val_v.at[pl.ds(0, BCH)])
                pltpu.sync_copy(val_v.at[pl.ds(0, BCH)],
                                shared.at[pl.ds(o, BCH)])

    def _drain_to(dst_hbm):
        for j in range((N_BCH + NS - 1) // NS):
            k = j * NS + s
            @pl.when(k < N_BCH)
            def _():
                o = k * BCH
                pltpu.sync_copy(shared.at[pl.ds(o, BCH)],
                                val_v.at[pl.ds(0, BCH)])
                pltpu.sync_copy(val_v.at[pl.ds(0, BCH)],
                                dst_hbm.at[pl.ds(lo + o, BCH)])

    @pl.when(c == 0)
    def _():
        pltpu.sync_copy(ida_hbm.at[pl.ds(base, PER_TILE)],
                        idx_v.at[pl.ds(0, PER_TILE)])

    @pl.when(c == 1)
    def _():
        pltpu.sync_copy(idb_hbm.at[pl.ds(base, PER_TILE)],
                        idx_v.at[pl.ds(0, PER_TILE)])

    _init_from(acc_in_hbm)
    pltpu.sync_copy(contrib_hbm.at[pl.ds(base, PER_TILE)],
                    val_v.at[pl.ds(0, PER_TILE)])
    plsc.subcore_barrier()

    def _scatter():
        def wstep(w, carry):
            sl = pl.ds(w * WIN, WIN)
            pltpu.sync_copy(val_v.at[sl], shared.at[idx_v.at[sl]], add=True)
            return carry
        lax.fori_loop(0, PER_TILE // WIN, wstep, 0)

    _scatter()
    plsc.subcore_barrier()
    _drain_to(acc_out_hbm)

    _init_from(den_in_hbm)

    def ones_step(i, carry):
        val_v[pl.ds(i * LANES, LANES)] = jnp.full((LANES,), 1.0, jnp.float32)
        return carry

    lax.fori_loop(0, PER_TILE // LANES, ones_step, 0)
    plsc.subcore_barrier()
    _scatter()
    plsc.subcore_barrier()
    _drain_to(den_out_hbm)


_sc_call = pl.kernel(
    _sc_body,
    out_type=(
        jax.ShapeDtypeStruct((N_MODEL_ROWS,), jnp.float32),
        jax.ShapeDtypeStruct((N_MODEL_ROWS,), jnp.float32),
    ),
    mesh=plsc.VectorSubcoreMesh(
        core_axis_name="c", subcore_axis_name="s",
        num_cores=NC, num_subcores=NS),
    scratch_types=[
        pltpu.VMEM((PER_TILE,), jnp.int32),
        pltpu.VMEM((PER_TILE,), jnp.float32),
        pltpu.VMEM_SHARED((ROWS_PER_CORE + DUMMY_ROWS,), jnp.float32),
    ],
)


@jax.jit
def kernel(viewspace_grad, update_filter, gaussian_ids, xyz_gradient_accum,
           denom):
    pad = N_PAD - N_POINTS
    x = jnp.pad(viewspace_grad[:, 0], (0, pad)).reshape(TC_ROWS, 128)
    y = jnp.pad(viewspace_grad[:, 1], (0, pad)).reshape(TC_ROWS, 128)
    m = jnp.pad(update_filter, (0, pad)).reshape(TC_ROWS, 128)
    ids = jnp.pad(gaussian_ids.astype(jnp.int32), (0, pad))
    contrib, ida, idb = _norm_call(x, y, m, ids.reshape(TC_ROWS, 128))
    acc, den = _sc_call(ida.reshape(N_PAD), idb.reshape(N_PAD),
                        contrib.reshape(N_PAD),
                        xyz_gradient_accum[:, 0], denom[:, 0])
    return jnp.stack([acc, den], axis=-1)


# speedup vs baseline: 12.8005x; 1.6357x over previous
import jax
import jax.numpy as jnp
from jax import lax
from jax.experimental import pallas as pl
from jax.experimental.pallas import tpu as pltpu
from jax.experimental.pallas import tpu_sc as plsc

N_MODEL_ROWS = 2_000_000
N_POINTS = 500_000
NC = 2
NS = 16
LANES = 16

N_PAD = 524_288
TC_ROWS = N_PAD // 128
TC_BLOCK = 1024
PER_TILE = N_PAD // NS
ROWS_PER_CORE = N_MODEL_ROWS // NC
BCH = 25_000
N_BCH = ROWS_PER_CORE // BCH
DUMMY_ROWS = 64
WIN = 2048


def _norm_body(x_ref, y_ref, m_ref, ids_ref, o_ref, ida_ref, idb_ref):
    xx = x_ref[...]
    yy = y_ref[...]
    m = m_ref[...]
    ids = ids_ref[...]
    o_ref[...] = jnp.sqrt(xx * xx + yy * yy) * m.astype(jnp.float32)
    sink = ROWS_PER_CORE + (
        lax.broadcasted_iota(jnp.int32, (TC_BLOCK, 128), 1) % DUMMY_ROWS)
    ida_ref[...] = jnp.where(m & (ids < ROWS_PER_CORE), ids, sink)
    idb_ref[...] = jnp.where(m & (ids >= ROWS_PER_CORE),
                             ids - ROWS_PER_CORE, sink)


_norm_call = pl.pallas_call(
    _norm_body,
    out_shape=(
        jax.ShapeDtypeStruct((TC_ROWS, 128), jnp.float32),
        jax.ShapeDtypeStruct((TC_ROWS, 128), jnp.int32),
        jax.ShapeDtypeStruct((TC_ROWS, 128), jnp.int32),
    ),
    grid=(TC_ROWS // TC_BLOCK,),
    in_specs=[pl.BlockSpec((TC_BLOCK, 128), lambda i: (i, 0))] * 4,
    out_specs=[pl.BlockSpec((TC_BLOCK, 128), lambda i: (i, 0))] * 3,
)


def _sc_body(ida_hbm, idb_hbm, contrib_hbm,
             acc_out_hbm, den_out_hbm, idx_v, val_v, shared):
    c = lax.axis_index("c")
    s = lax.axis_index("s")
    base = s * PER_TILE
    lo = c * ROWS_PER_CORE

    def _fill(value):
        def fstep(i, carry):
            val_v[pl.ds(i * LANES, LANES)] = jnp.full((LANES,), value,
                                                      jnp.float32)
            return carry
        lax.fori_loop(0, BCH // LANES, fstep, 0)
        val_v[pl.ds(BCH - LANES, LANES)] = jnp.full((LANES,), value,
                                                    jnp.float32)

    def _zero_init():
        _fill(0.0)
        for j in range((N_BCH + NS - 1) // NS):
            k = j * NS + s
            @pl.when(k < N_BCH)
            def _():
                o = k * BCH
                pltpu.sync_copy(val_v.at[pl.ds(0, BCH)],
                                shared.at[pl.ds(o, BCH)])

    def _drain_to(dst_hbm):
        for j in range((N_BCH + NS - 1) // NS):
            k = j * NS + s
            @pl.when(k < N_BCH)
            def _():
                o = k * BCH
                pltpu.sync_copy(shared.at[pl.ds(o, BCH)],
                                val_v.at[pl.ds(0, BCH)])
                pltpu.sync_copy(val_v.at[pl.ds(0, BCH)],
                                dst_hbm.at[pl.ds(lo + o, BCH)])

    @pl.when(c == 0)
    def _():
        pltpu.sync_copy(ida_hbm.at[pl.ds(base, PER_TILE)],
                        idx_v.at[pl.ds(0, PER_TILE)])

    @pl.when(c == 1)
    def _():
        pltpu.sync_copy(idb_hbm.at[pl.ds(base, PER_TILE)],
                        idx_v.at[pl.ds(0, PER_TILE)])

    _zero_init()
    pltpu.sync_copy(contrib_hbm.at[pl.ds(base, PER_TILE)],
                    val_v.at[pl.ds(0, PER_TILE)])
    plsc.subcore_barrier()

    def _scatter():
        def wstep(w, carry):
            sl = pl.ds(w * WIN, WIN)
            pltpu.sync_copy(val_v.at[sl], shared.at[idx_v.at[sl]], add=True)
            return carry
        lax.fori_loop(0, PER_TILE // WIN, wstep, 0)

    _scatter()
    plsc.subcore_barrier()
    _drain_to(acc_out_hbm)

    _zero_init()

    def ones_step(i, carry):
        val_v[pl.ds(i * LANES, LANES)] = jnp.full((LANES,), 1.0, jnp.float32)
        return carry

    lax.fori_loop(0, PER_TILE // LANES, ones_step, 0)
    plsc.subcore_barrier()
    _scatter()
    plsc.subcore_barrier()
    _drain_to(den_out_hbm)


_sc_call = pl.kernel(
    _sc_body,
    out_type=(
        jax.ShapeDtypeStruct((N_MODEL_ROWS,), jnp.float32),
        jax.ShapeDtypeStruct((N_MODEL_ROWS,), jnp.float32),
    ),
    mesh=plsc.VectorSubcoreMesh(
        core_axis_name="c", subcore_axis_name="s",
        num_cores=NC, num_subcores=NS),
    scratch_types=[
        pltpu.VMEM((PER_TILE,), jnp.int32),
        pltpu.VMEM((PER_TILE,), jnp.float32),
        pltpu.VMEM_SHARED((ROWS_PER_CORE + DUMMY_ROWS,), jnp.float32),
    ],
)


@jax.jit
def kernel(viewspace_grad, update_filter, gaussian_ids, xyz_gradient_accum,
           denom):
    pad = N_PAD - N_POINTS
    x = jnp.pad(viewspace_grad[:, 0], (0, pad)).reshape(TC_ROWS, 128)
    y = jnp.pad(viewspace_grad[:, 1], (0, pad)).reshape(TC_ROWS, 128)
    m = jnp.pad(update_filter, (0, pad)).reshape(TC_ROWS, 128)
    ids = jnp.pad(gaussian_ids.astype(jnp.int32), (0, pad))
    contrib, ida, idb = _norm_call(x, y, m, ids.reshape(TC_ROWS, 128))
    acc, den = _sc_call(ida.reshape(N_PAD), idb.reshape(N_PAD),
                        contrib.reshape(N_PAD))
    return (jnp.stack([acc, den], axis=-1)
            + jnp.concatenate([xyz_gradient_accum, denom], axis=-1))
